# baseline (speedup 1.0000x reference)
import functools

import jax
import jax.numpy as jnp
from jax.experimental import pallas as pl

D_MODEL = 1024
D_FF = 4096
N_PATTERNS = 128
TOPK = 8
TOKEN_BLOCK = 256


def _ffn_body(x_ref, r_ref, xbf_ref, patterns_ref, gates_ref, pw_ref, pb_ref,
              upw_ref, upb_ref, dww_ref, dwb_ref, out_ref):
    xb = x_ref[...]
    rb = r_ref[...]

    pat = patterns_ref[...]
    ps = jax.lax.dot_general(xb, pat, (((1,), (1,)), ((), ())),
                             preferred_element_type=jnp.float32)
    rs = jax.lax.dot_general(rb, pat, (((1,), (1,)), ((), ())),
                             preferred_element_type=jnp.float32)

    pw = pw_ref[...]
    l0 = (jnp.sum(xb * pw[0:1, :D_MODEL], axis=1)
          + jnp.sum(rb * pw[0:1, D_MODEL:], axis=1) + pb_ref[0, 0])
    l1 = (jnp.sum(xb * pw[1:2, :D_MODEL], axis=1)
          + jnp.sum(rb * pw[1:2, D_MODEL:], axis=1) + pb_ref[0, 1])
    w0 = jax.nn.sigmoid(l0 - l1)[:, None]
    scores = w0 * ps + (1.0 - w0) * rs

    lanes = jax.lax.broadcasted_iota(jnp.int32, scores.shape, 1)
    s = scores
    vals = []
    onehots = []
    for _ in range(TOPK):
        m = jnp.max(s, axis=1, keepdims=True)
        idx = jnp.min(jnp.where(s == m, lanes, N_PATTERNS), axis=1,
                      keepdims=True)
        sel = lanes == idx
        vals.append(m)
        onehots.append(sel)
        s = jnp.where(sel, -jnp.inf, s)

    exps = [jnp.exp(v - vals[0]) for v in vals]
    denom = exps[0]
    for e in exps[1:]:
        denom = denom + e
    wmat = jnp.zeros_like(scores)
    for e, sel in zip(exps, onehots):
        wmat = wmat + jnp.where(sel, e / denom, 0.0)

    ffn_gate = jax.lax.dot_general(wmat.astype(jnp.bfloat16), gates_ref[...],
                                   (((1,), (0,)), ((), ())),
                                   preferred_element_type=jnp.float32)

    h = jax.lax.dot_general(xbf_ref[...], upw_ref[...],
                            (((1,), (1,)), ((), ())),
                            preferred_element_type=jnp.float32)
    h = h + upb_ref[...]
    h = h * jax.nn.sigmoid(ffn_gate)
    h = 0.5 * h * (1.0 + jax.lax.erf(h * 0.7071067811865476))
    out = jax.lax.dot_general(h.astype(jnp.bfloat16), dww_ref[...],
                              (((1,), (1,)), ((), ())),
                              preferred_element_type=jnp.float32)
    out_ref[...] = out + dwb_ref[...]


@functools.partial(jax.jit, static_argnames=())
def kernel(x, router_out, patterns, gates, path_w, path_b, up_w, up_b,
           down_w, down_b):
    B, S, _ = x.shape
    x2 = x.reshape(B * S, D_MODEL)
    r2 = router_out.reshape(B * S, D_MODEL)
    xbf = x2.astype(jnp.bfloat16)
    gates_bf = gates.astype(jnp.bfloat16)
    upw_bf = up_w.astype(jnp.bfloat16)
    dww_bf = down_w.astype(jnp.bfloat16)
    pb2 = path_b.reshape(1, 2)
    upb2 = up_b.reshape(1, D_FF)
    dwb2 = down_b.reshape(1, D_MODEL)

    n_blocks = (B * S) // TOKEN_BLOCK
    full = lambda shape: pl.BlockSpec(shape, lambda i: (0,) * len(shape))
    out = pl.pallas_call(
        _ffn_body,
        grid=(n_blocks,),
        in_specs=[
            pl.BlockSpec((TOKEN_BLOCK, D_MODEL), lambda i: (i, 0)),
            pl.BlockSpec((TOKEN_BLOCK, D_MODEL), lambda i: (i, 0)),
            pl.BlockSpec((TOKEN_BLOCK, D_MODEL), lambda i: (i, 0)),
            full((N_PATTERNS, D_MODEL)),
            full((N_PATTERNS, D_FF)),
            full((2, 2 * D_MODEL)),
            full((1, 2)),
            full((D_FF, D_MODEL)),
            full((1, D_FF)),
            full((D_MODEL, D_FF)),
            full((1, D_MODEL)),
        ],
        out_specs=pl.BlockSpec((TOKEN_BLOCK, D_MODEL), lambda i: (i, 0)),
        out_shape=jax.ShapeDtypeStruct((B * S, D_MODEL), jnp.float32),
    )(x2, r2, xbf, patterns, gates_bf, path_w, pb2, upw_bf, upb2, dww_bf,
      dwb2)
    return out.reshape(B, S, D_MODEL)


# speedup vs baseline: 1.4169x; 1.4169x over previous
import functools

import jax
import jax.numpy as jnp
from jax.experimental import pallas as pl

D_MODEL = 1024
D_FF = 4096
N_PATTERNS = 128
TOPK = 8
TOKEN_BLOCK = 512


def _ffn_body(x_ref, r_ref, patterns_ref, gates_ref, pw_ref, pb_ref,
              upw_ref, upb_ref, dww_ref, dwb_ref, out_ref):
    xb = x_ref[...]
    rb = r_ref[...]

    pat = patterns_ref[...]
    ps = jax.lax.dot_general(xb, pat, (((1,), (1,)), ((), ())),
                             preferred_element_type=jnp.float32)
    rs = jax.lax.dot_general(rb, pat, (((1,), (1,)), ((), ())),
                             preferred_element_type=jnp.float32)

    pw = pw_ref[...]
    l0 = (jnp.sum(xb * pw[0:1, :D_MODEL], axis=1)
          + jnp.sum(rb * pw[0:1, D_MODEL:], axis=1) + pb_ref[0, 0])
    l1 = (jnp.sum(xb * pw[1:2, :D_MODEL], axis=1)
          + jnp.sum(rb * pw[1:2, D_MODEL:], axis=1) + pb_ref[0, 1])
    w0 = jax.nn.sigmoid(l0 - l1)[:, None]
    scores = w0 * ps + (1.0 - w0) * rs

    lanes = jax.lax.broadcasted_iota(jnp.int32, scores.shape, 1)
    s = scores
    vals = []
    onehots = []
    for _ in range(TOPK):
        m = jnp.max(s, axis=1, keepdims=True)
        idx = jnp.min(jnp.where(s == m, lanes, N_PATTERNS), axis=1,
                      keepdims=True)
        sel = lanes == idx
        vals.append(m)
        onehots.append(sel)
        s = jnp.where(sel, -jnp.inf, s)

    exps = [jnp.exp(v - vals[0]) for v in vals]
    denom = exps[0]
    for e in exps[1:]:
        denom = denom + e
    wmat = jnp.zeros_like(scores)
    for e, sel in zip(exps, onehots):
        wmat = wmat + jnp.where(sel, e / denom, 0.0)

    ffn_gate = jax.lax.dot_general(wmat, gates_ref[...],
                                   (((1,), (0,)), ((), ())),
                                   preferred_element_type=jnp.float32)

    h = jax.lax.dot_general(xb, upw_ref[...], (((1,), (1,)), ((), ())),
                            preferred_element_type=jnp.float32)
    h = h + upb_ref[...]
    h = h * jax.nn.sigmoid(ffn_gate)
    h = 0.5 * h * (1.0 + jax.lax.erf(h * 0.7071067811865476))
    out = jax.lax.dot_general(h, dww_ref[...], (((1,), (1,)), ((), ())),
                              preferred_element_type=jnp.float32)
    out_ref[...] = out + dwb_ref[...]


@functools.partial(jax.jit, static_argnames=())
def kernel(x, router_out, patterns, gates, path_w, path_b, up_w, up_b,
           down_w, down_b):
    B, S, _ = x.shape
    x2 = x.reshape(B * S, D_MODEL)
    r2 = router_out.reshape(B * S, D_MODEL)
    pb2 = path_b.reshape(1, 2)
    upb2 = up_b.reshape(1, D_FF)
    dwb2 = down_b.reshape(1, D_MODEL)

    n_blocks = (B * S) // TOKEN_BLOCK
    full = lambda shape: pl.BlockSpec(shape, lambda i: (0,) * len(shape))
    out = pl.pallas_call(
        _ffn_body,
        grid=(n_blocks,),
        in_specs=[
            pl.BlockSpec((TOKEN_BLOCK, D_MODEL), lambda i: (i, 0)),
            pl.BlockSpec((TOKEN_BLOCK, D_MODEL), lambda i: (i, 0)),
            full((N_PATTERNS, D_MODEL)),
            full((N_PATTERNS, D_FF)),
            full((2, 2 * D_MODEL)),
            full((1, 2)),
            full((D_FF, D_MODEL)),
            full((1, D_FF)),
            full((D_MODEL, D_FF)),
            full((1, D_MODEL)),
        ],
        out_specs=pl.BlockSpec((TOKEN_BLOCK, D_MODEL), lambda i: (i, 0)),
        out_shape=jax.ShapeDtypeStruct((B * S, D_MODEL), jnp.float32),
    )(x2, r2, patterns, gates, path_w, pb2, up_w, upb2, down_w, dwb2)
    return out.reshape(B, S, D_MODEL)
